# Initial kernel scaffold; baseline (speedup 1.0000x reference)
import jax
import jax.numpy as jnp
from jax.experimental import pallas as pl


def kernel(node_position, center_idx):
    raise NotImplementedError("write your pallas kernel here")


# speedup vs baseline: 1.1714x; 1.1714x over previous
import functools

import jax
import jax.numpy as jnp
from jax import lax
from jax.experimental import pallas as pl
from jax.experimental.pallas import tpu as pltpu
from jax.experimental.pallas import tpu_sc as plsc

B = 50
N = 2000
K = 50
NB = N // 16
NW = 32
MIN_RADIUS = 15.0


def _gather(ref, idx):
    return plsc.load_gather(ref, [idx])


def _worker_id():
    return lax.axis_index("s") * 2 + lax.axis_index("c")


def _sc_body(pos_hbm, cidx_hbm, s_out, sk_out, posv, sv, cidxv, skv):
    wid = _worker_id()
    pltpu.sync_copy(cidx_hbm, cidxv)
    lane3 = lax.iota(jnp.int32, 16) * 3

    def do_graph(b):
        pltpu.sync_copy(pos_hbm.at[b], posv)
        bvec = jnp.full((16,), b, jnp.int32)
        c3 = _gather(cidxv, bvec) * 3
        cx = _gather(posv, c3)
        cy = _gather(posv, c3 + 1)
        cz = _gather(posv, c3 + 2)

        def dist_body(j, carry):
            idx = lane3 + j * 48
            dx = _gather(posv, idx) - cx
            dy = _gather(posv, idx + 1) - cy
            dz = _gather(posv, idx + 2) - cz
            s = ((dx * dx + dy * dy) + dz * dz) + 1e-12
            sv[pl.ds(j * 16, 16)] = s
            return carry

        lax.fori_loop(0, NB, dist_body, 0)

        def bit_body(i, cbits):
            cand = cbits | lax.shift_left(jnp.int32(1), jnp.int32(30) - i)
            candv = jnp.full((16,), cand, jnp.int32)

            def cnt_body(j, acc):
                u = lax.bitcast_convert_type(sv[pl.ds(j * 16, 16)], jnp.int32)
                return acc + jnp.where(u < candv, jnp.int32(1), jnp.int32(0))

            acc = lax.fori_loop(0, NB, cnt_body, jnp.zeros((16,), jnp.int32))
            total = jnp.sum(acc)
            return jnp.where(total >= K, cbits, cand)

        kbits = lax.fori_loop(0, 31, bit_body, jnp.int32(0))
        skv[...] = lax.bitcast_convert_type(
            jnp.full((16,), kbits, jnp.int32), jnp.float32)
        pltpu.sync_copy(sv, s_out.at[b])
        pltpu.sync_copy(skv, sk_out.at[b])

    do_graph(wid)

    @pl.when(wid + NW < B)
    def _():
        do_graph(wid + NW)


@jax.jit
def _sc_stage(pos_flat, cidx_padded):
    mesh = plsc.VectorSubcoreMesh(core_axis_name="c", subcore_axis_name="s")
    f = pl.kernel(
        _sc_body,
        out_type=[
            jax.ShapeDtypeStruct((B, N), jnp.float32),
            jax.ShapeDtypeStruct((B, 16), jnp.float32),
        ],
        mesh=mesh,
        compiler_params=pltpu.CompilerParams(needs_layout_passes=False),
        scratch_types=[
            pltpu.VMEM((N * 3,), jnp.float32),
            pltpu.VMEM((N,), jnp.float32),
            pltpu.VMEM((64,), jnp.int32),
            pltpu.VMEM((16,), jnp.float32),
        ],
    )
    return f(pos_flat, cidx_padded)


def _tc_body(s_ref, sk_ref, md_ref, mask_ref, rad_ref):
    s = s_ref[...]
    dist = jnp.sqrt(s)
    dk = jnp.sqrt(sk_ref[...])
    radius = jnp.maximum(dk * jnp.float32(1.5), jnp.float32(MIN_RADIUS))
    rad_ref[...] = radius
    rb = jnp.broadcast_to(radius[:, 0:1], (B, N))
    mask = dist < rb
    md_ref[...] = jnp.where(mask, dist, jnp.float32(0.0))
    mask_ref[...] = mask


@jax.jit
def _tc_stage(s, sk):
    return pl.pallas_call(
        _tc_body,
        out_shape=(
            jax.ShapeDtypeStruct((B, N), jnp.float32),
            jax.ShapeDtypeStruct((B, N), jnp.bool_),
            jax.ShapeDtypeStruct((B, 16), jnp.float32),
        ),
    )(s, sk)


def kernel(node_position, center_idx):
    pos_flat = node_position.reshape(B, N * 3)
    cidx = jnp.pad(center_idx.astype(jnp.int32), (0, 64 - B))
    s, sk = _sc_stage(pos_flat, cidx)
    md, mask, rad = _tc_stage(s, sk)
    return md, mask, rad[:, 0]


# speedup vs baseline: 2.5070x; 2.1402x over previous
import jax
import jax.numpy as jnp
from jax import lax
from jax.experimental import pallas as pl
from jax.experimental.pallas import tpu as pltpu
from jax.experimental.pallas import tpu_sc as plsc

B = 50
N = 2000
K = 50
NB = N // 16
NW = 32
SLOTS = 128
MIN_RADIUS = 15.0
INF = float("inf")


def _gather(ref, idx):
    return plsc.load_gather(ref, [idx])


def _scatter(ref, idx, x, mask):
    plsc.store_scatter(ref, [idx], x, mask=mask)


def _copy(src, dst):
    pltpu.sync_copy(src, dst)


def _worker_id():
    return lax.axis_index("s") * 2 + lax.axis_index("c")


def _sc_body(x_hbm, y_hbm, z_hbm, cidx_hbm, s_out, sk_out,
             xv, yv, zv, sv, candv, cidxv, skv, sem):
    wid = _worker_id()
    _copy(cidx_hbm, cidxv)
    lane = lax.iota(jnp.int32, 16)
    lanebase = lane * SLOTS
    zero16 = jnp.zeros((16,), jnp.int32)
    one16 = jnp.ones((16,), jnp.int32)
    infv = jnp.full((16,), INF, jnp.float32)

    def start_copies(b):
        cp1 = pltpu.make_async_copy(x_hbm.at[b], xv, sem)
        cp2 = pltpu.make_async_copy(y_hbm.at[b], yv, sem)
        cp3 = pltpu.make_async_copy(z_hbm.at[b], zv, sem)
        cp1.start(); cp2.start(); cp3.start()

    def do_graph(b, prefetch_next):
        cp1 = pltpu.make_async_copy(x_hbm.at[b], xv, sem)
        cp2 = pltpu.make_async_copy(y_hbm.at[b], yv, sem)
        cp3 = pltpu.make_async_copy(z_hbm.at[b], zv, sem)
        cp1.wait(); cp2.wait(); cp3.wait()
        bvec = jnp.full((16,), b, jnp.int32)
        cidx16 = _gather(cidxv, bvec)
        cx = _gather(xv, cidx16)
        cy = _gather(yv, cidx16)
        cz = _gather(zv, cidx16)

        def dist_body(i, ms):
            ms = list(ms)
            for u in range(5):
                j = i * 5 + u
                dx = xv[pl.ds(j * 16, 16)] - cx
                dy = yv[pl.ds(j * 16, 16)] - cy
                dz = zv[pl.ds(j * 16, 16)] - cz
                s = ((dx * dx + dy * dy) + dz * dz) + 1e-12
                sv[pl.ds(j * 16, 16)] = s
                for t in range(7, 0, -1):
                    ms[t] = jnp.minimum(ms[t], jnp.maximum(ms[t - 1], s))
                ms[0] = jnp.minimum(ms[0], s)
            return tuple(ms)

        ms = lax.fori_loop(0, NB // 5, dist_body, (infv,) * 8)
        @pl.when(prefetch_next)
        def _():
            start_copies(b + NW)

        us = [lax.bitcast_convert_type(m, jnp.int32) for m in ms]
        kvec = jnp.full((16,), K, jnp.int32)

        def t1_body(i, cbits):
            cand = cbits | lax.shift_left(
                jnp.full((16,), 1, jnp.int32),
                jnp.full((16,), 30, jnp.int32) - i)
            total = zero16
            for u in us:
                total = total + plsc.all_reduce_population_count(u < cand)
            return jnp.where(total >= kvec, cbits, cand)

        t1bits = lax.fori_loop(0, 31, t1_body, zero16)
        t1v = lax.bitcast_convert_type(t1bits, jnp.float32)

        def count_body(i, carry):
            cless, cleq = carry
            for u in range(5):
                s = sv[pl.ds((i * 5 + u) * 16, 16)]
                cless = cless + plsc.all_reduce_population_count(s < t1v)
                cleq = cleq + plsc.all_reduce_population_count(s <= t1v)
            return (cless, cleq)

        cless, cleq = lax.fori_loop(
            0, NB // 5, count_body, (zero16, zero16))
        verified = jnp.logical_and(cless[0] <= K - 1, cleq[0] >= K)

        def slow_path():
            def fill_body(i, carry):
                for u in range(8):
                    candv[pl.ds((i * 8 + u) * 16, 16)] = infv
                return carry

            lax.fori_loop(0, SLOTS // 8, fill_body, 0)

            def compact_body(i, cnt):
                ss = [sv[pl.ds((i * 5 + u) * 16, 16)] for u in range(5)]
                les = [s <= t1v for s in ss]
                incs = [jnp.where(m, one16, zero16) for m in les]
                offs = [cnt]
                for u in range(4):
                    offs.append(offs[-1] + incs[u])
                for u in range(5):
                    _scatter(candv, lanebase + offs[u], ss[u], les[u])
                return offs[4] + incs[4]

            cnt = lax.fori_loop(0, NB // 5, compact_body, zero16)
            ns2 = (lax.reduce_max(cnt, axes=(0,)) + 1) // 2

            def bit_body(i, cbits):
                cand = cbits | lax.shift_left(
                    jnp.full((16,), 1, jnp.int32),
                    jnp.full((16,), 30, jnp.int32) - i)

                def cnt_body(j, acc):
                    u0 = lax.bitcast_convert_type(
                        _gather(candv, lanebase + 2 * j), jnp.int32)
                    u1 = lax.bitcast_convert_type(
                        _gather(candv, lanebase + 2 * j + 1), jnp.int32)
                    c0 = plsc.all_reduce_population_count(u0 < cand)
                    c1 = plsc.all_reduce_population_count(u1 < cand)
                    return acc + c0 + c1

                total = lax.fori_loop(0, ns2, cnt_body, zero16)
                return jnp.where(total >= kvec, cbits, cand)

            return lax.fori_loop(0, 31, bit_body, zero16)

        kbits = lax.cond(verified, lambda: t1bits, slow_path)
        skv[...] = lax.bitcast_convert_type(kbits, jnp.float32)
        _copy(sv, s_out.at[b])
        _copy(skv, sk_out.at[b])

    ngraphs = 1 + jnp.where(wid + NW < B, 1, 0)
    start_copies(wid)

    def graph_body(g, carry):
        do_graph(wid + NW * g, g + 1 < ngraphs)
        return carry

    lax.fori_loop(0, ngraphs, graph_body, 0)


@jax.jit
def _sc_stage(xp, yp, zp, cidx):
    mesh = plsc.VectorSubcoreMesh(core_axis_name="c", subcore_axis_name="s")
    f = pl.kernel(
        _sc_body,
        out_type=[
            jax.ShapeDtypeStruct((B, N), jnp.float32),
            jax.ShapeDtypeStruct((B, 16), jnp.float32),
        ],
        mesh=mesh,
        compiler_params=pltpu.CompilerParams(needs_layout_passes=False),
        scratch_types=[
            pltpu.VMEM((N,), jnp.float32),
            pltpu.VMEM((N,), jnp.float32),
            pltpu.VMEM((N,), jnp.float32),
            pltpu.VMEM((N,), jnp.float32),
            pltpu.VMEM((16 * SLOTS,), jnp.float32),
            pltpu.VMEM((B,), jnp.int32),
            pltpu.VMEM((16,), jnp.float32),
            pltpu.SemaphoreType.DMA,
        ],
    )
    return f(xp, yp, zp, cidx)


def _tc_body(s_ref, sk_ref, md_ref, mask_ref, rad_ref):
    s = s_ref[...]
    dist = jnp.sqrt(s)
    dk = jnp.sqrt(sk_ref[...])
    radius = jnp.maximum(dk * jnp.float32(1.5), jnp.float32(MIN_RADIUS))
    rad_ref[...] = radius[:, 0]
    rb = jnp.broadcast_to(radius[:, 0:1], (B, N))
    mask = dist < rb
    md_ref[...] = jnp.where(mask, dist, jnp.float32(0.0))
    mask_ref[...] = mask.astype(jnp.int8)


@jax.jit
def _tc_stage(s, sk):
    return pl.pallas_call(
        _tc_body,
        out_shape=(
            jax.ShapeDtypeStruct((B, N), jnp.float32),
            jax.ShapeDtypeStruct((B, N), jnp.int8),
            jax.ShapeDtypeStruct((B,), jnp.float32),
        ),
    )(s, sk)


def kernel(node_position, center_idx):
    xp = node_position[:, :, 0]
    yp = node_position[:, :, 1]
    zp = node_position[:, :, 2]
    s, sk = _sc_stage(xp, yp, zp, center_idx.astype(jnp.int32))
    md, mask8, rad = _tc_stage(s, sk)
    return md, mask8.astype(jnp.bool_), rad
